# baseline (speedup 1.0000x reference)
import functools

import jax
import jax.numpy as jnp
from jax import lax
from jax.experimental import pallas as pl
from jax.experimental.pallas import tpu as pltpu
from jax.experimental.pallas import tpu_sc as plsc

N = 10000
E = 320000
D = 128
H = 256
C = 10
G = 128

NC = 2
NS = 16
LANES = 16

K = 128
EP = 327680
CHUNKS = EP // K
NP = 10240
RPT = NP // NS
DUMP = NP - 1

_mesh = functools.partial(plsc.VectorSubcoreMesh,
                          core_axis_name="c", subcore_axis_name="s")


def _sc_aggregate(table0, table1, src2d, dst2d, zsrc, *, edge_split,
                  gather=True):
    if edge_split:
        cpt = CHUNKS // (NC * NS)

        def tile_c0(cid, sid):
            return (cid * NS + sid) * cpt
    else:
        cpt = CHUNKS // NS

        def tile_c0(cid, sid):
            del cid
            return sid * cpt

    @functools.partial(
        pl.kernel, mesh=_mesh(),
        out_type=jax.ShapeDtypeStruct((NC, NP, D), jnp.float32),
        scratch_types=[
            pltpu.VMEM((K,), jnp.int32),
            pltpu.VMEM((K,), jnp.int32),
            pltpu.VMEM((K,), jnp.int32),
            pltpu.VMEM((K,), jnp.int32),
            pltpu.VMEM((K, D), jnp.float32),
            pltpu.VMEM((K, D), jnp.float32),
            pltpu.VMEM_SHARED((NP, D), jnp.float32),
            pltpu.SemaphoreType.DMA,
            pltpu.SemaphoreType.DMA,
            pltpu.SemaphoreType.DMA,
            pltpu.SemaphoreType.DMA,
            pltpu.SemaphoreType.DMA,
            pltpu.SemaphoreType.DMA,
        ],
    )
    def k(t0_hbm, t1_hbm, src_hbm, dst_hbm, z_hbm, out_hbm,
          sidx0, didx0, sidx1, didx1, rows0, rows1, acc,
          is0, id0, is1, id1, g0, g1):
        cid = lax.axis_index("c")
        sid = lax.axis_index("s")
        pltpu.sync_copy(z_hbm, acc.at[pl.ds(sid * RPT, RPT)])
        plsc.subcore_barrier()
        c0 = tile_c0(cid, sid)

        sl = ((sidx0, didx0, rows0, is0, id0, g0),
              (sidx1, didx1, rows1, is1, id1, g1))

        def istart(c, s):
            si, di, _, isem, idsem, _ = sl[s]
            pltpu.make_async_copy(
                src_hbm.at[pl.ds((c0 + c) * K, K)], si, isem).start()
            pltpu.make_async_copy(
                dst_hbm.at[pl.ds((c0 + c) * K, K)], di, idsem).start()

        def iwait(s):
            si, di, _, isem, idsem, _ = sl[s]
            pltpu.make_async_copy(src_hbm.at[pl.ds(0, K)], si, isem).wait()
            pltpu.make_async_copy(dst_hbm.at[pl.ds(0, K)], di, idsem).wait()

        def gstart(s):
            si, _, rw, _, _, gsem = sl[s]

            @pl.when(cid == 0)
            def _():
                pltpu.make_async_copy(t0_hbm.at[si], rw, gsem).start()

            @pl.when(cid == 1)
            def _():
                pltpu.make_async_copy(t1_hbm.at[si], rw, gsem).start()

        def gwait_scatter(s):
            _, di, rw, _, _, gsem = sl[s]
            pltpu.make_async_copy(t0_hbm.at[di], rw, gsem).wait()
            pltpu.sync_copy(rw, acc.at[di], add=True)

        def dstart(c, s):
            _, di, _, _, idsem, _ = sl[s]
            pltpu.make_async_copy(
                dst_hbm.at[pl.ds((c0 + c) * K, K)], di, idsem).start()

        def dwait_scatter(s):
            _, di, _, _, idsem, _ = sl[s]
            pltpu.make_async_copy(dst_hbm.at[pl.ds(0, K)], di, idsem).wait()
            pltpu.sync_copy(rows0, acc.at[di], add=True)

        if gather:
            istart(0, 0)
            istart(1, 1)
            iwait(0)
            gstart(0)

            @pl.loop(0, cpt, step=2)
            def _(c):
                iwait(1)
                gstart(1)
                gwait_scatter(0)

                @pl.when(c + 2 < cpt)
                def _():
                    istart(c + 2, 0)

                gwait_scatter(1)

                @pl.when(c + 3 < cpt)
                def _():
                    istart(c + 3, 1)

                @pl.when(c + 2 < cpt)
                def _():
                    iwait(0)
                    gstart(0)
        else:
            pltpu.sync_copy(t0_hbm.at[pl.ds(0, K)], rows0)
            dstart(0, 0)
            dstart(1, 1)

            @pl.loop(0, cpt, step=2)
            def _(c):
                dwait_scatter(0)

                @pl.when(c + 2 < cpt)
                def _():
                    dstart(c + 2, 0)

                dwait_scatter(1)

                @pl.when(c + 3 < cpt)
                def _():
                    dstart(c + 3, 1)

        plsc.subcore_barrier()
        pltpu.sync_copy(acc.at[pl.ds(sid * RPT, RPT)],
                        out_hbm.at[cid].at[pl.ds(sid * RPT, RPT)])

    return k(table0, table1, src2d, dst2d, zsrc)


_HIGHEST = jax.lax.Precision.HIGHEST


def _dot(a, b):
    return jax.lax.dot_general(a, b, (((1,), (0,)), ((), ())),
                               precision=_HIGHEST,
                               preferred_element_type=jnp.float32)


def _tc_prescale(deg2, x):
    BLK = 2000

    def body(deg_ref, x_ref, dinv_ref, xs_ref, xs2_ref):
        counts = deg_ref[0, :, :1] + deg_ref[1, :, :1]
        dv = jax.lax.rsqrt(counts + 1.0)
        dinv_ref[...] = dv
        xs = x_ref[...] * dv
        xs_ref[...] = xs
        xs2_ref[...] = xs

    return pl.pallas_call(
        body,
        grid=(N // BLK,),
        in_specs=[pl.BlockSpec((NC, BLK, D), lambda i: (0, i, 0)),
                  pl.BlockSpec((BLK, D), lambda i: (i, 0))],
        out_specs=[pl.BlockSpec((BLK, 1), lambda i: (i, 0)),
                   pl.BlockSpec((BLK, D), lambda i: (i, 0)),
                   pl.BlockSpec((BLK, D), lambda i: (i, 0))],
        out_shape=[jax.ShapeDtypeStruct((N, 1), jnp.float32),
                   jax.ShapeDtypeStruct((N, D), jnp.float32),
                   jax.ShapeDtypeStruct((N, D), jnp.float32)],
    )(deg2, x)


def _tc_layer1(agg2, xs, dinv, W1, b1r):
    BLK = 1000

    def body(a_ref, xs_ref, dv_ref, w_ref, b_ref, lo_ref, hi_ref):
        dv = dv_ref[...]
        t = (a_ref[0] + a_ref[1] + xs_ref[...]) * dv
        h = jnp.maximum(_dot(t, w_ref[...]) + b_ref[...], 0.0)
        hs = h * dv
        lo_ref[...] = hs[:, :D]
        hi_ref[...] = hs[:, D:]

    return pl.pallas_call(
        body,
        grid=(N // BLK,),
        in_specs=[pl.BlockSpec((NC, BLK, D), lambda i: (0, i, 0)),
                  pl.BlockSpec((BLK, D), lambda i: (i, 0)),
                  pl.BlockSpec((BLK, 1), lambda i: (i, 0)),
                  pl.BlockSpec((D, H), lambda i: (0, 0)),
                  pl.BlockSpec((1, H), lambda i: (0, 0))],
        out_specs=[pl.BlockSpec((BLK, D), lambda i: (i, 0)),
                   pl.BlockSpec((BLK, D), lambda i: (i, 0))],
        out_shape=[jax.ShapeDtypeStruct((N, D), jnp.float32),
                   jax.ShapeDtypeStruct((N, D), jnp.float32)],
    )(agg2, xs, dinv, W1, b1r)


def _tc_final(agg4, h1lo, h1hi, dinv, batch2d, W2, b2r, Wf1, bf1r, Wf2, bf2r):
    BLK = 1000
    NBLK = N // BLK

    def body(a_ref, lo_ref, hi_ref, dv_ref, b_ref, w2_ref, b2_ref,
             wf1_ref, bf1_ref, wf2_ref, bf2_ref, out_ref, sums, cnts):
        i = pl.program_id(0)
        dv = dv_ref[...]
        tlo = (a_ref[0] + lo_ref[...]) * dv
        thi = (a_ref[1] + hi_ref[...]) * dv
        w2 = w2_ref[...]
        h2 = jnp.maximum(
            _dot(tlo, w2[:D, :]) + _dot(thi, w2[D:, :]) + b2_ref[...], 0.0)
        ids = jax.lax.broadcasted_iota(jnp.int32, (BLK, G), 1)
        oh = (b_ref[...] == ids).astype(jnp.float32)
        pooled_blk = jax.lax.dot_general(
            oh, h2, (((0,), (0,)), ((), ())),
            precision=_HIGHEST, preferred_element_type=jnp.float32)
        cnt_blk = jax.lax.dot_general(
            oh, jnp.ones((BLK, 1), jnp.float32), (((0,), (0,)), ((), ())),
            precision=_HIGHEST, preferred_element_type=jnp.float32)

        @pl.when(i == 0)
        def _():
            sums[...] = jnp.zeros_like(sums)
            cnts[...] = jnp.zeros_like(cnts)

        sums[...] += pooled_blk
        cnts[...] += cnt_blk

        @pl.when(i == NBLK - 1)
        def _():
            pooled = sums[...] / jnp.maximum(cnts[...], 1.0)
            h3 = jnp.maximum(_dot(pooled, wf1_ref[...]) + bf1_ref[...], 0.0)
            out_ref[...] = _dot(h3, wf2_ref[...]) + bf2_ref[...]

    return pl.pallas_call(
        body,
        grid=(NBLK,),
        in_specs=[pl.BlockSpec((NC, BLK, D), lambda i: (0, i, 0)),
                  pl.BlockSpec((BLK, D), lambda i: (i, 0)),
                  pl.BlockSpec((BLK, D), lambda i: (i, 0)),
                  pl.BlockSpec((BLK, 1), lambda i: (i, 0)),
                  pl.BlockSpec((BLK, 1), lambda i: (i, 0)),
                  pl.BlockSpec((H, H), lambda i: (0, 0)),
                  pl.BlockSpec((1, H), lambda i: (0, 0)),
                  pl.BlockSpec((H, H), lambda i: (0, 0)),
                  pl.BlockSpec((1, H), lambda i: (0, 0)),
                  pl.BlockSpec((H, C), lambda i: (0, 0)),
                  pl.BlockSpec((1, C), lambda i: (0, 0))],
        out_specs=pl.BlockSpec((G, C), lambda i: (0, 0)),
        out_shape=jax.ShapeDtypeStruct((G, C), jnp.float32),
        scratch_shapes=[pltpu.VMEM((G, H), jnp.float32),
                        pltpu.VMEM((G, 1), jnp.float32)],
    )(agg4, h1lo, h1hi, dinv, batch2d, W2, b2r, Wf1, bf1r, Wf2, bf2r)


def kernel(x, edge_index, batch, W1, b1, W2, b2, Wf1, bf1, Wf2, bf2):
    pad = EP - E
    src1d = jnp.concatenate([edge_index[0], jnp.zeros((pad,), jnp.int32)])
    dst1d = jnp.concatenate(
        [edge_index[1], jnp.full((pad,), DUMP, jnp.int32)])
    zsrc128 = jnp.zeros((RPT, D), jnp.float32)
    ones_nd = jnp.ones((N, D), jnp.float32)
    ones_nd2 = jnp.full((N, D), 1.0, jnp.float32)

    deg2 = _sc_aggregate(ones_nd, ones_nd2, src1d, dst1d, zsrc128,
                         edge_split=True)
    dinv, xs, xs2 = _tc_prescale(deg2, x)
    agg2 = _sc_aggregate(xs, xs2, src1d, dst1d, zsrc128, edge_split=True)
    h1lo, h1hi = _tc_layer1(agg2, xs, dinv, W1, b1.reshape(1, H))
    agg4 = _sc_aggregate(h1lo, h1hi, src1d, dst1d, zsrc128,
                         edge_split=False)
    return _tc_final(agg4, h1lo, h1hi, dinv, batch.reshape(N, 1), W2,
                     b2.reshape(1, H), Wf1, bf1.reshape(1, H), Wf2,
                     bf2.reshape(1, C))


# speedup vs baseline: 1.0145x; 1.0145x over previous
import functools

import jax
import jax.numpy as jnp
from jax import lax
from jax.experimental import pallas as pl
from jax.experimental.pallas import tpu as pltpu
from jax.experimental.pallas import tpu_sc as plsc

N = 10000
E = 320000
D = 128
H = 256
C = 10
G = 128

NC = 2
NS = 16
LANES = 16

K = 128
EP = 327680
CHUNKS = EP // K
NP = 10240
RPT = NP // NS
DUMP = NP - 1

_mesh = functools.partial(plsc.VectorSubcoreMesh,
                          core_axis_name="c", subcore_axis_name="s")


def _sc_aggregate(table0, table1, src2d, dst2d, zsrc, *, edge_split,
                  gather=True):
    if edge_split:
        cpt = CHUNKS // (NC * NS)

        def tile_c0(cid, sid):
            return (cid * NS + sid) * cpt
    else:
        cpt = CHUNKS // NS

        def tile_c0(cid, sid):
            del cid
            return sid * cpt

    @functools.partial(
        pl.kernel, mesh=_mesh(),
        out_type=jax.ShapeDtypeStruct((NC, NP, D), jnp.float32),
        scratch_types=[
            pltpu.VMEM((K,), jnp.int32),
            pltpu.VMEM((K,), jnp.int32),
            pltpu.VMEM((K,), jnp.int32),
            pltpu.VMEM((K,), jnp.int32),
            pltpu.VMEM((K, D), jnp.float32),
            pltpu.VMEM((K, D), jnp.float32),
            pltpu.VMEM_SHARED((NP, D), jnp.float32),
            pltpu.SemaphoreType.DMA,
            pltpu.SemaphoreType.DMA,
            pltpu.SemaphoreType.DMA,
            pltpu.SemaphoreType.DMA,
            pltpu.SemaphoreType.DMA,
            pltpu.SemaphoreType.DMA,
            pltpu.SemaphoreType.DMA,
            pltpu.SemaphoreType.DMA,
        ],
    )
    def k(t0_hbm, t1_hbm, src_hbm, dst_hbm, z_hbm, out_hbm,
          sidx0, didx0, sidx1, didx1, rows0, rows1, acc,
          is0, id0, is1, id1, g0, g1, s0, s1):
        cid = lax.axis_index("c")
        sid = lax.axis_index("s")
        pltpu.sync_copy(z_hbm, acc.at[pl.ds(sid * RPT, RPT)])
        plsc.subcore_barrier()
        c0 = tile_c0(cid, sid)

        sl = ((sidx0, didx0, rows0, is0, id0, g0, s0),
              (sidx1, didx1, rows1, is1, id1, g1, s1))

        def istart(c, s):
            si, di, _, isem, idsem, _, _ = sl[s]
            pltpu.make_async_copy(
                src_hbm.at[pl.ds((c0 + c) * K, K)], si, isem).start()
            pltpu.make_async_copy(
                dst_hbm.at[pl.ds((c0 + c) * K, K)], di, idsem).start()

        def iwait(s):
            si, di, _, isem, idsem, _, _ = sl[s]
            pltpu.make_async_copy(src_hbm.at[pl.ds(0, K)], si, isem).wait()
            pltpu.make_async_copy(dst_hbm.at[pl.ds(0, K)], di, idsem).wait()

        def gstart(s):
            si, _, rw, _, _, gsem, _ = sl[s]

            @pl.when(cid == 0)
            def _():
                pltpu.make_async_copy(t0_hbm.at[si], rw, gsem).start()

            @pl.when(cid == 1)
            def _():
                pltpu.make_async_copy(t1_hbm.at[si], rw, gsem).start()

        def gwait_scatter(s):
            _, di, rw, _, _, gsem, ssem = sl[s]
            pltpu.make_async_copy(t0_hbm.at[di], rw, gsem).wait()
            pltpu.make_async_copy(rw, acc.at[di], ssem).start()

        def swait(s):
            _, di, rw, _, _, _, ssem = sl[s]
            pltpu.make_async_copy(t0_hbm.at[di], rw, ssem).wait()

        def dstart(c, s):
            _, di, _, _, idsem, _, _ = sl[s]
            pltpu.make_async_copy(
                dst_hbm.at[pl.ds((c0 + c) * K, K)], di, idsem).start()

        def dwait_scatter(s):
            _, di, _, _, idsem, _, _ = sl[s]
            pltpu.make_async_copy(dst_hbm.at[pl.ds(0, K)], di, idsem).wait()
            pltpu.sync_copy(rows0, acc.at[di], add=True)

        if gather:
            istart(0, 0)
            istart(1, 1)
            iwait(0)
            gstart(0)

            @pl.loop(0, cpt, step=2)
            def _(c):
                iwait(1)
                gstart(1)
                gwait_scatter(0)

                gwait_scatter(1)

                @pl.when(c + 2 < cpt)
                def _():
                    swait(0)
                    istart(c + 2, 0)

                @pl.when(c + 3 < cpt)
                def _():
                    swait(1)
                    istart(c + 3, 1)

                @pl.when(c + 2 < cpt)
                def _():
                    iwait(0)
                    gstart(0)
        else:
            pltpu.sync_copy(t0_hbm.at[pl.ds(0, K)], rows0)
            dstart(0, 0)
            dstart(1, 1)

            @pl.loop(0, cpt, step=2)
            def _(c):
                dwait_scatter(0)

                @pl.when(c + 2 < cpt)
                def _():
                    dstart(c + 2, 0)

                dwait_scatter(1)

                @pl.when(c + 3 < cpt)
                def _():
                    dstart(c + 3, 1)

        if gather:
            swait(0)
            swait(1)
        plsc.subcore_barrier()
        pltpu.sync_copy(acc.at[pl.ds(sid * RPT, RPT)],
                        out_hbm.at[cid].at[pl.ds(sid * RPT, RPT)])

    return k(table0, table1, src2d, dst2d, zsrc)


_HIGHEST = jax.lax.Precision.HIGHEST


def _dot(a, b):
    return jax.lax.dot_general(a, b, (((1,), (0,)), ((), ())),
                               precision=_HIGHEST,
                               preferred_element_type=jnp.float32)


def _tc_prescale(deg2, x):
    BLK = 2000

    def body(deg_ref, x_ref, dinv_ref, xs_ref):
        counts = deg_ref[0, :, :1] + deg_ref[1, :, :1]
        dv = jax.lax.rsqrt(counts + 1.0)
        dinv_ref[...] = dv
        xs_ref[...] = x_ref[...] * dv

    return pl.pallas_call(
        body,
        grid=(N // BLK,),
        in_specs=[pl.BlockSpec((NC, BLK, D), lambda i: (0, i, 0)),
                  pl.BlockSpec((BLK, D), lambda i: (i, 0))],
        out_specs=[pl.BlockSpec((BLK, 1), lambda i: (i, 0)),
                   pl.BlockSpec((BLK, D), lambda i: (i, 0))],
        out_shape=[jax.ShapeDtypeStruct((N, 1), jnp.float32),
                   jax.ShapeDtypeStruct((N, D), jnp.float32)],
    )(deg2, x)


def _tc_layer1(agg2, xs, dinv, W1, b1r):
    BLK = 1000

    def body(a_ref, xs_ref, dv_ref, w_ref, b_ref, lo_ref, hi_ref):
        dv = dv_ref[...]
        t = (a_ref[0] + a_ref[1] + xs_ref[...]) * dv
        h = jnp.maximum(_dot(t, w_ref[...]) + b_ref[...], 0.0)
        hs = h * dv
        lo_ref[...] = hs[:, :D]
        hi_ref[...] = hs[:, D:]

    return pl.pallas_call(
        body,
        grid=(N // BLK,),
        in_specs=[pl.BlockSpec((NC, BLK, D), lambda i: (0, i, 0)),
                  pl.BlockSpec((BLK, D), lambda i: (i, 0)),
                  pl.BlockSpec((BLK, 1), lambda i: (i, 0)),
                  pl.BlockSpec((D, H), lambda i: (0, 0)),
                  pl.BlockSpec((1, H), lambda i: (0, 0))],
        out_specs=[pl.BlockSpec((BLK, D), lambda i: (i, 0)),
                   pl.BlockSpec((BLK, D), lambda i: (i, 0))],
        out_shape=[jax.ShapeDtypeStruct((N, D), jnp.float32),
                   jax.ShapeDtypeStruct((N, D), jnp.float32)],
    )(agg2, xs, dinv, W1, b1r)


def _tc_final(agg4, h1lo, h1hi, dinv, batch2d, W2, b2r, Wf1, bf1r, Wf2, bf2r):
    BLK = 1000
    NBLK = N // BLK

    def body(a_ref, lo_ref, hi_ref, dv_ref, b_ref, w2_ref, b2_ref,
             wf1_ref, bf1_ref, wf2_ref, bf2_ref, out_ref, sums, cnts):
        i = pl.program_id(0)
        dv = dv_ref[...]
        tlo = (a_ref[0] + lo_ref[...]) * dv
        thi = (a_ref[1] + hi_ref[...]) * dv
        w2 = w2_ref[...]
        h2 = jnp.maximum(
            _dot(tlo, w2[:D, :]) + _dot(thi, w2[D:, :]) + b2_ref[...], 0.0)
        ids = jax.lax.broadcasted_iota(jnp.int32, (BLK, G), 1)
        oh = (b_ref[...] == ids).astype(jnp.float32)
        pooled_blk = jax.lax.dot_general(
            oh, h2, (((0,), (0,)), ((), ())),
            precision=_HIGHEST, preferred_element_type=jnp.float32)
        cnt_blk = jax.lax.dot_general(
            oh, jnp.ones((BLK, 1), jnp.float32), (((0,), (0,)), ((), ())),
            precision=_HIGHEST, preferred_element_type=jnp.float32)

        @pl.when(i == 0)
        def _():
            sums[...] = jnp.zeros_like(sums)
            cnts[...] = jnp.zeros_like(cnts)

        sums[...] += pooled_blk
        cnts[...] += cnt_blk

        @pl.when(i == NBLK - 1)
        def _():
            pooled = sums[...] / jnp.maximum(cnts[...], 1.0)
            h3 = jnp.maximum(_dot(pooled, wf1_ref[...]) + bf1_ref[...], 0.0)
            out_ref[...] = _dot(h3, wf2_ref[...]) + bf2_ref[...]

    return pl.pallas_call(
        body,
        grid=(NBLK,),
        in_specs=[pl.BlockSpec((NC, BLK, D), lambda i: (0, i, 0)),
                  pl.BlockSpec((BLK, D), lambda i: (i, 0)),
                  pl.BlockSpec((BLK, D), lambda i: (i, 0)),
                  pl.BlockSpec((BLK, 1), lambda i: (i, 0)),
                  pl.BlockSpec((BLK, 1), lambda i: (i, 0)),
                  pl.BlockSpec((H, H), lambda i: (0, 0)),
                  pl.BlockSpec((1, H), lambda i: (0, 0)),
                  pl.BlockSpec((H, H), lambda i: (0, 0)),
                  pl.BlockSpec((1, H), lambda i: (0, 0)),
                  pl.BlockSpec((H, C), lambda i: (0, 0)),
                  pl.BlockSpec((1, C), lambda i: (0, 0))],
        out_specs=pl.BlockSpec((G, C), lambda i: (0, 0)),
        out_shape=jax.ShapeDtypeStruct((G, C), jnp.float32),
        scratch_shapes=[pltpu.VMEM((G, H), jnp.float32),
                        pltpu.VMEM((G, 1), jnp.float32)],
    )(agg4, h1lo, h1hi, dinv, batch2d, W2, b2r, Wf1, bf1r, Wf2, bf2r)


def kernel(x, edge_index, batch, W1, b1, W2, b2, Wf1, bf1, Wf2, bf2):
    pad = EP - E
    src1d = jnp.concatenate([edge_index[0], jnp.zeros((pad,), jnp.int32)])
    dst1d = jnp.concatenate(
        [edge_index[1], jnp.full((pad,), DUMP, jnp.int32)])
    zsrc128 = jnp.zeros((RPT, D), jnp.float32)
    ones_nd = jnp.ones((N, D), jnp.float32)

    deg2 = _sc_aggregate(ones_nd, ones_nd, src1d, dst1d, zsrc128,
                         edge_split=True)
    dinv, xs = _tc_prescale(deg2, x)
    agg2 = _sc_aggregate(xs, xs, src1d, dst1d, zsrc128, edge_split=True)
    h1lo, h1hi = _tc_layer1(agg2, xs, dinv, W1, b1.reshape(1, H))
    agg4 = _sc_aggregate(h1lo, h1hi, src1d, dst1d, zsrc128,
                         edge_split=False)
    return _tc_final(agg4, h1lo, h1hi, dinv, batch.reshape(N, 1), W2,
                     b2.reshape(1, H), Wf1, bf1.reshape(1, H), Wf2,
                     bf2.reshape(1, C))


# speedup vs baseline: 1.0307x; 1.0160x over previous
import functools

import jax
import jax.numpy as jnp
from jax import lax
from jax.experimental import pallas as pl
from jax.experimental.pallas import tpu as pltpu
from jax.experimental.pallas import tpu_sc as plsc

N = 10000
E = 320000
D = 128
H = 256
C = 10
G = 128

NC = 2
NS = 16
LANES = 16

K = 128
EP = 327680
CHUNKS = EP // K
NP = 10240
RPT = NP // NS
DUMP = NP - 1

_mesh = functools.partial(plsc.VectorSubcoreMesh,
                          core_axis_name="c", subcore_axis_name="s")


def _sc_aggregate(table0, table1, src2d, dst2d, zsrc, *, edge_split,
                  gather=True):
    if edge_split:
        cpt = CHUNKS // (NC * NS)

        def tile_c0(cid, sid):
            return (cid * NS + sid) * cpt
    else:
        cpt = CHUNKS // NS

        def tile_c0(cid, sid):
            del cid
            return sid * cpt

    @functools.partial(
        pl.kernel, mesh=_mesh(),
        out_type=jax.ShapeDtypeStruct((NC, NP, D), jnp.float32),
        scratch_types=[
            pltpu.VMEM((K,), jnp.int32),
            pltpu.VMEM((K,), jnp.int32),
            pltpu.VMEM((K,), jnp.int32),
            pltpu.VMEM((K,), jnp.int32),
            pltpu.VMEM((K, D), jnp.float32),
            pltpu.VMEM((K, D), jnp.float32),
            pltpu.VMEM_SHARED((NP, D), jnp.float32),
            pltpu.SemaphoreType.DMA,
            pltpu.SemaphoreType.DMA,
            pltpu.SemaphoreType.DMA,
            pltpu.SemaphoreType.DMA,
            pltpu.SemaphoreType.DMA,
            pltpu.SemaphoreType.DMA,
            pltpu.SemaphoreType.DMA,
            pltpu.SemaphoreType.DMA,
        ],
    )
    def k(t0_hbm, t1_hbm, src_hbm, dst_hbm, z_hbm, out_hbm,
          sidx0, didx0, sidx1, didx1, rows0, rows1, acc,
          is0, id0, is1, id1, g0, g1, s0, s1):
        cid = lax.axis_index("c")
        sid = lax.axis_index("s")
        pltpu.sync_copy(z_hbm, acc.at[pl.ds(sid * RPT, RPT)])
        plsc.subcore_barrier()
        c0 = tile_c0(cid, sid)

        sl = ((sidx0, didx0, rows0, is0, id0, g0, s0),
              (sidx1, didx1, rows1, is1, id1, g1, s1))

        def istart(c, s):
            si, di, _, isem, idsem, _, _ = sl[s]
            pltpu.make_async_copy(
                src_hbm.at[pl.ds((c0 + c) * K, K)], si, isem).start()
            pltpu.make_async_copy(
                dst_hbm.at[pl.ds((c0 + c) * K, K)], di, idsem).start()

        def iwait(s):
            si, di, _, isem, idsem, _, _ = sl[s]
            pltpu.make_async_copy(src_hbm.at[pl.ds(0, K)], si, isem).wait()
            pltpu.make_async_copy(dst_hbm.at[pl.ds(0, K)], di, idsem).wait()

        def gstart(s):
            si, _, rw, _, _, gsem, _ = sl[s]

            @pl.when(cid == 0)
            def _():
                pltpu.make_async_copy(t0_hbm.at[si], rw, gsem).start()

            @pl.when(cid == 1)
            def _():
                pltpu.make_async_copy(t1_hbm.at[si], rw, gsem).start()

        def gwait_scatter(s):
            _, di, rw, _, _, gsem, ssem = sl[s]
            del ssem
            pltpu.make_async_copy(t0_hbm.at[di], rw, gsem).wait()
            pltpu.sync_copy(rw, acc.at[di], add=True)

        def dstart(c, s):
            _, di, _, _, idsem, _, _ = sl[s]
            pltpu.make_async_copy(
                dst_hbm.at[pl.ds((c0 + c) * K, K)], di, idsem).start()

        def dwait_scatter(s):
            _, di, _, _, idsem, _, _ = sl[s]
            pltpu.make_async_copy(dst_hbm.at[pl.ds(0, K)], di, idsem).wait()
            pltpu.sync_copy(rows0, acc.at[di], add=True)

        if gather:
            istart(0, 0)
            istart(1, 1)
            iwait(0)
            gstart(0)

            @pl.loop(0, cpt, step=2)
            def _(c):
                iwait(1)
                gstart(1)
                gwait_scatter(0)

                @pl.when(c + 2 < cpt)
                def _():
                    istart(c + 2, 0)

                gwait_scatter(1)

                @pl.when(c + 3 < cpt)
                def _():
                    istart(c + 3, 1)

                @pl.when(c + 2 < cpt)
                def _():
                    iwait(0)
                    gstart(0)
        else:
            pltpu.sync_copy(t0_hbm.at[pl.ds(0, K)], rows0)
            dstart(0, 0)
            dstart(1, 1)

            @pl.loop(0, cpt, step=2)
            def _(c):
                dwait_scatter(0)

                @pl.when(c + 2 < cpt)
                def _():
                    dstart(c + 2, 0)

                dwait_scatter(1)

                @pl.when(c + 3 < cpt)
                def _():
                    dstart(c + 3, 1)

        plsc.subcore_barrier()
        pltpu.sync_copy(acc.at[pl.ds(sid * RPT, RPT)],
                        out_hbm.at[cid].at[pl.ds(sid * RPT, RPT)])

    return k(table0, table1, src2d, dst2d, zsrc)


_HIGHEST = jax.lax.Precision.HIGHEST


def _dot(a, b):
    return jax.lax.dot_general(a, b, (((1,), (0,)), ((), ())),
                               precision=_HIGHEST,
                               preferred_element_type=jnp.float32)


def _tc_prescale(deg2, x):
    BLK = 2000

    def body(deg_ref, x_ref, dinv_ref, xs_ref):
        counts = deg_ref[0, :, :1] + deg_ref[1, :, :1]
        dv = jax.lax.rsqrt(counts + 1.0)
        dinv_ref[...] = dv
        xs_ref[...] = x_ref[...] * dv

    return pl.pallas_call(
        body,
        grid=(N // BLK,),
        in_specs=[pl.BlockSpec((NC, BLK, D), lambda i: (0, i, 0)),
                  pl.BlockSpec((BLK, D), lambda i: (i, 0))],
        out_specs=[pl.BlockSpec((BLK, 1), lambda i: (i, 0)),
                   pl.BlockSpec((BLK, D), lambda i: (i, 0))],
        out_shape=[jax.ShapeDtypeStruct((N, 1), jnp.float32),
                   jax.ShapeDtypeStruct((N, D), jnp.float32)],
    )(deg2, x)


def _tc_layer1(agg2, xs, dinv, W1, b1r):
    BLK = 1000

    def body(a_ref, xs_ref, dv_ref, w_ref, b_ref, lo_ref, hi_ref):
        dv = dv_ref[...]
        t = (a_ref[0] + a_ref[1] + xs_ref[...]) * dv
        h = jnp.maximum(_dot(t, w_ref[...]) + b_ref[...], 0.0)
        hs = h * dv
        lo_ref[...] = hs[:, :D]
        hi_ref[...] = hs[:, D:]

    return pl.pallas_call(
        body,
        grid=(N // BLK,),
        in_specs=[pl.BlockSpec((NC, BLK, D), lambda i: (0, i, 0)),
                  pl.BlockSpec((BLK, D), lambda i: (i, 0)),
                  pl.BlockSpec((BLK, 1), lambda i: (i, 0)),
                  pl.BlockSpec((D, H), lambda i: (0, 0)),
                  pl.BlockSpec((1, H), lambda i: (0, 0))],
        out_specs=[pl.BlockSpec((BLK, D), lambda i: (i, 0)),
                   pl.BlockSpec((BLK, D), lambda i: (i, 0))],
        out_shape=[jax.ShapeDtypeStruct((N, D), jnp.float32),
                   jax.ShapeDtypeStruct((N, D), jnp.float32)],
    )(agg2, xs, dinv, W1, b1r)


def _tc_final(agg4, h1lo, h1hi, dinv, batch2d, W2, b2r, Wf1, bf1r, Wf2, bf2r):
    BLK = 1000
    NBLK = N // BLK

    def body(a_ref, lo_ref, hi_ref, dv_ref, b_ref, w2_ref, b2_ref,
             wf1_ref, bf1_ref, wf2_ref, bf2_ref, out_ref, sums, cnts):
        i = pl.program_id(0)
        dv = dv_ref[...]
        tlo = (a_ref[0] + lo_ref[...]) * dv
        thi = (a_ref[1] + hi_ref[...]) * dv
        w2 = w2_ref[...]
        h2 = jnp.maximum(
            _dot(tlo, w2[:D, :]) + _dot(thi, w2[D:, :]) + b2_ref[...], 0.0)
        ids = jax.lax.broadcasted_iota(jnp.int32, (BLK, G), 1)
        oh = (b_ref[...] == ids).astype(jnp.float32)
        pooled_blk = jax.lax.dot_general(
            oh, h2, (((0,), (0,)), ((), ())),
            precision=_HIGHEST, preferred_element_type=jnp.float32)
        cnt_blk = jax.lax.dot_general(
            oh, jnp.ones((BLK, 1), jnp.float32), (((0,), (0,)), ((), ())),
            precision=_HIGHEST, preferred_element_type=jnp.float32)

        @pl.when(i == 0)
        def _():
            sums[...] = jnp.zeros_like(sums)
            cnts[...] = jnp.zeros_like(cnts)

        sums[...] += pooled_blk
        cnts[...] += cnt_blk

        @pl.when(i == NBLK - 1)
        def _():
            pooled = sums[...] / jnp.maximum(cnts[...], 1.0)
            h3 = jnp.maximum(_dot(pooled, wf1_ref[...]) + bf1_ref[...], 0.0)
            out_ref[...] = _dot(h3, wf2_ref[...]) + bf2_ref[...]

    return pl.pallas_call(
        body,
        grid=(NBLK,),
        in_specs=[pl.BlockSpec((NC, BLK, D), lambda i: (0, i, 0)),
                  pl.BlockSpec((BLK, D), lambda i: (i, 0)),
                  pl.BlockSpec((BLK, D), lambda i: (i, 0)),
                  pl.BlockSpec((BLK, 1), lambda i: (i, 0)),
                  pl.BlockSpec((BLK, 1), lambda i: (i, 0)),
                  pl.BlockSpec((H, H), lambda i: (0, 0)),
                  pl.BlockSpec((1, H), lambda i: (0, 0)),
                  pl.BlockSpec((H, H), lambda i: (0, 0)),
                  pl.BlockSpec((1, H), lambda i: (0, 0)),
                  pl.BlockSpec((H, C), lambda i: (0, 0)),
                  pl.BlockSpec((1, C), lambda i: (0, 0))],
        out_specs=pl.BlockSpec((G, C), lambda i: (0, 0)),
        out_shape=jax.ShapeDtypeStruct((G, C), jnp.float32),
        scratch_shapes=[pltpu.VMEM((G, H), jnp.float32),
                        pltpu.VMEM((G, 1), jnp.float32)],
    )(agg4, h1lo, h1hi, dinv, batch2d, W2, b2r, Wf1, bf1r, Wf2, bf2r)


def kernel(x, edge_index, batch, W1, b1, W2, b2, Wf1, bf1, Wf2, bf2):
    pad = EP - E
    src1d = jnp.concatenate([edge_index[0], jnp.zeros((pad,), jnp.int32)])
    dst1d = jnp.concatenate(
        [edge_index[1], jnp.full((pad,), DUMP, jnp.int32)])
    zsrc128 = jnp.zeros((RPT, D), jnp.float32)
    ones_nd = jnp.ones((N, D), jnp.float32)

    deg2 = _sc_aggregate(ones_nd, ones_nd, src1d, dst1d, zsrc128,
                         edge_split=True)
    dinv, xs = _tc_prescale(deg2, x)
    agg2 = _sc_aggregate(xs, xs, src1d, dst1d, zsrc128, edge_split=True)
    h1lo, h1hi = _tc_layer1(agg2, xs, dinv, W1, b1.reshape(1, H))
    agg4 = _sc_aggregate(h1lo, h1hi, src1d, dst1d, zsrc128,
                         edge_split=False)
    return _tc_final(agg4, h1lo, h1hi, dinv, batch.reshape(N, 1), W2,
                     b2.reshape(1, H), Wf1, bf1.reshape(1, H), Wf2,
                     bf2.reshape(1, C))
